# baseline (speedup 1.0000x reference)
import functools

import jax
import jax.numpy as jnp
from jax import lax
from jax.experimental import pallas as pl
from jax.experimental.pallas import tpu as pltpu
from jax.experimental.pallas import tpu_sc as plsc

B, N, K, D = 4096, 200, 50, 128
PADK = 56
NC, NS, L = 2, 16, 16
NW = NC * NS
ROWS = B * K
BPW = B // NW
RPW = ROWS // NW
PPW = BPW * PADK
BPC = 2
CHUNK = BPC * PADK
NCHUNK = BPW // BPC
NB = 6

_MAGIC = 9363
_SHIFT = 16

_mesh = plsc.VectorSubcoreMesh(
    core_axis_name="c", subcore_axis_name="s", num_cores=NC, num_subcores=NS
)


@functools.partial(
    pl.kernel,
    out_type=jax.ShapeDtypeStruct((B * PADK, D), jnp.float32),
    mesh=_mesh,
    scratch_types=[
        pltpu.VMEM((RPW,), jnp.int32),
        pltpu.VMEM((PPW,), jnp.int32),
        pltpu.VMEM((NB, CHUNK, D), jnp.float32),
    ]
    + [pltpu.SemaphoreType.DMA] * (2 * NB),
    compiler_params=pltpu.CompilerParams(
        needs_layout_passes=False, use_tc_tiling_on_sc=True
    ),
)
def _gather(src_hbm, idx_hbm, out_hbm, idx_raw, idx_pad, rows, *sems):
    sem_g = sems[:NB]
    sem_o = sems[NB:]
    wid = lax.axis_index("s") * NC + lax.axis_index("c")
    wbase = wid * RPW
    bbase = wid * BPW
    lane = lax.iota(jnp.int32, L)

    pltpu.sync_copy(idx_hbm.at[pl.ds(wbase, RPW)], idx_raw)

    def build_pad_chunk(g):
        for u in range(CHUNK // L):
            p = g * CHUNK + u * L + lane
            j = lax.shift_right_logical(
                lax.shift_right_logical(p, 3) * _MAGIC, _SHIFT
            )
            k = p - j * PADK
            t = j * K + jnp.minimum(k, K - 1)
            raw = plsc.load_gather(idx_raw, [t])
            idx_pad[pl.ds(g * CHUNK + u * L, L)] = (bbase + j) * N + raw

    gathers = {}
    writes = {}

    def start_gather(g):
        gathers[g] = pltpu.async_copy(
            src_hbm.at[idx_pad.at[pl.ds(g * CHUNK, CHUNK)]],
            rows.at[g % NB],
            sem_g[g % NB],
        )

    def start_write(g):
        writes[g] = pltpu.async_copy(
            rows.at[g % NB],
            out_hbm.at[pl.ds((bbase + g * BPC) * PADK, CHUNK)],
            sem_o[g % NB],
        )

    for g in range(NCHUNK):
        build_pad_chunk(g)
        if g >= NB:
            writes[g - NB].wait()
        start_gather(g)
        if g >= NB - 1:
            gathers[g - (NB - 1)].wait()
            start_write(g - (NB - 1))
    for g in range(NCHUNK - (NB - 1), NCHUNK):
        gathers[g].wait()
        start_write(g)
    for g in range(NCHUNK - NB, NCHUNK):
        writes[g].wait()


def kernel(source, idxs):
    src = source.reshape(B * N, D)
    idx = idxs.astype(jnp.int32).reshape(ROWS)
    out = _gather(src, idx)
    return out.reshape(B, PADK, D)[:, :K, :]


# speedup vs baseline: 1.1977x; 1.1977x over previous
import functools

import jax
import jax.numpy as jnp
from jax import lax
from jax.experimental import pallas as pl
from jax.experimental.pallas import tpu as pltpu
from jax.experimental.pallas import tpu_sc as plsc

B, N, K, D = 4096, 200, 50, 128
PADK = 56
NC, NS, L = 2, 16, 16
NW = NC * NS
ROWS = B * K
BPW = B // NW
RPW = ROWS // NW
PPW = BPW * PADK
BPC = 2
NCHUNK = BPW // BPC
NB = 6

_MAGIC = 9363
_SHIFT = 16

_mesh = plsc.VectorSubcoreMesh(
    core_axis_name="c", subcore_axis_name="s", num_cores=NC, num_subcores=NS
)


@functools.partial(
    pl.kernel,
    out_type=jax.ShapeDtypeStruct((B, K, D), jnp.float32),
    mesh=_mesh,
    scratch_types=[
        pltpu.VMEM((RPW,), jnp.int32),
        pltpu.VMEM((PPW,), jnp.int32),
        pltpu.VMEM((NB, BPC, K, D), jnp.float32),
    ]
    + [pltpu.SemaphoreType.DMA] * (2 * NB),
    compiler_params=pltpu.CompilerParams(
        needs_layout_passes=False, use_tc_tiling_on_sc=True
    ),
)
def _gather(src_hbm, idx_hbm, out_hbm, idx_raw, idx_pad, rows, *sems):
    sem_g = sems[:NB]
    sem_o = sems[NB:]
    wid = lax.axis_index("s") * NC + lax.axis_index("c")
    wbase = wid * RPW
    bbase = wid * BPW
    lane = lax.iota(jnp.int32, L)

    pltpu.sync_copy(idx_hbm.at[pl.ds(wbase, RPW)], idx_raw)

    def build_pad_chunk(g):
        for u in range(BPC * PADK // L):
            p = g * BPC * PADK + u * L + lane
            j = lax.shift_right_logical(
                lax.shift_right_logical(p, 3) * _MAGIC, _SHIFT
            )
            k = p - j * PADK
            t = j * K + jnp.minimum(k, K - 1)
            raw = plsc.load_gather(idx_raw, [t])
            idx_pad[pl.ds(g * BPC * PADK + u * L, L)] = (bbase + j) * N + raw

    gathers = {}
    writes = {}

    def start_gather(g):
        descs = gathers[g] = []
        for j in range(BPC):
            descs.append(
                pltpu.async_copy(
                    src_hbm.at[
                        idx_pad.at[pl.ds((g * BPC + j) * PADK, K)]
                    ],
                    rows.at[g % NB].at[j],
                    sem_g[g % NB],
                )
            )

    def start_write(g):
        writes[g] = pltpu.async_copy(
            rows.at[g % NB],
            out_hbm.at[pl.ds(bbase + g * BPC, BPC)],
            sem_o[g % NB],
        )

    for g in range(NCHUNK):
        build_pad_chunk(g)
        if g >= NB:
            writes[g - NB].wait()
        start_gather(g)
        if g >= NB - 1:
            for d in gathers[g - (NB - 1)]:
                d.wait()
            start_write(g - (NB - 1))
    for g in range(NCHUNK - (NB - 1), NCHUNK):
        for d in gathers[g]:
            d.wait()
        start_write(g)
    for g in range(NCHUNK - NB, NCHUNK):
        writes[g].wait()


def kernel(source, idxs):
    src = source.reshape(B * N, D)
    idx = idxs.astype(jnp.int32).reshape(ROWS)
    return _gather(src, idx)


# speedup vs baseline: 2.0925x; 1.7471x over previous
import functools

import jax
import jax.numpy as jnp
from jax import lax
from jax.experimental import pallas as pl
from jax.experimental.pallas import tpu as pltpu
from jax.experimental.pallas import tpu_sc as plsc

B, N, K, D = 4096, 200, 50, 128
NC, NS, L = 2, 16, 16
NW = NC * NS
ROWS = B * K
RPW = ROWS // NW
CHUNK = 128
NCHUNK = RPW // CHUNK
NB = 6

_mesh = plsc.VectorSubcoreMesh(
    core_axis_name="c", subcore_axis_name="s", num_cores=NC, num_subcores=NS
)


@functools.partial(
    pl.kernel,
    out_type=jax.ShapeDtypeStruct((ROWS, D), jnp.float32),
    mesh=_mesh,
    scratch_types=[
        pltpu.VMEM((RPW,), jnp.int32),
        pltpu.VMEM((NB, CHUNK, D), jnp.float32),
    ]
    + [pltpu.SemaphoreType.DMA] * (2 * NB),
    compiler_params=pltpu.CompilerParams(
        needs_layout_passes=False, use_tc_tiling_on_sc=True
    ),
)
def _gather(src_hbm, idx_hbm, out_hbm, idx_v, rows, *sems):
    sem_g = sems[:NB]
    sem_o = sems[NB:]
    wid = lax.axis_index("s") * NC + lax.axis_index("c")
    wbase = wid * RPW
    lane = lax.iota(jnp.int32, L)

    pltpu.sync_copy(idx_hbm.at[pl.ds(wbase, RPW)], idx_v)

    def flatten_chunk(g):
        for u in range(CHUNK // L):
            off = g * CHUNK + u * L
            b = lax.bitwise_and(wbase + off + lane, B - 1)
            idx_v[pl.ds(off, L)] = b * N + idx_v[pl.ds(off, L)]

    gathers = {}
    writes = {}

    def start_gather(g):
        gathers[g] = pltpu.async_copy(
            src_hbm.at[idx_v.at[pl.ds(g * CHUNK, CHUNK)]],
            rows.at[g % NB],
            sem_g[g % NB],
        )

    def start_write(g):
        writes[g] = pltpu.async_copy(
            rows.at[g % NB],
            out_hbm.at[pl.ds(wbase + g * CHUNK, CHUNK)],
            sem_o[g % NB],
        )

    for g in range(NCHUNK):
        flatten_chunk(g)
        if g >= NB:
            writes[g - NB].wait()
        start_gather(g)
        if g >= NB - 1:
            gathers[g - (NB - 1)].wait()
            start_write(g - (NB - 1))
    for g in range(NCHUNK - (NB - 1), NCHUNK):
        gathers[g].wait()
        start_write(g)
    for g in range(NCHUNK - NB, NCHUNK):
        writes[g].wait()


def kernel(source, idxs):
    src = source.reshape(B * N, D)
    idx_kmajor = idxs.astype(jnp.int32)[..., 0].T.reshape(ROWS)
    out = _gather(src, idx_kmajor)
    return out.reshape(K, B, D).transpose(1, 0, 2)


# speedup vs baseline: 2.1092x; 1.0080x over previous
import functools

import jax
import jax.numpy as jnp
from jax import lax
from jax.experimental import pallas as pl
from jax.experimental.pallas import tpu as pltpu
from jax.experimental.pallas import tpu_sc as plsc

B, N, K, D = 4096, 200, 50, 128
NC, NS, L = 2, 16, 16
NW = NC * NS
ROWS = B * K
RPW = ROWS // NW
CHUNK = 128
NCHUNK = RPW // CHUNK
NB = 7

_mesh = plsc.VectorSubcoreMesh(
    core_axis_name="c", subcore_axis_name="s", num_cores=NC, num_subcores=NS
)


@functools.partial(
    pl.kernel,
    out_type=jax.ShapeDtypeStruct((ROWS, D), jnp.float32),
    mesh=_mesh,
    scratch_types=[
        pltpu.VMEM((RPW,), jnp.int32),
        pltpu.VMEM((NB, CHUNK, D), jnp.float32),
    ]
    + [pltpu.SemaphoreType.DMA] * (2 * NB),
    compiler_params=pltpu.CompilerParams(
        needs_layout_passes=False, use_tc_tiling_on_sc=True
    ),
)
def _gather(src_hbm, idx_hbm, out_hbm, idx_v, rows, *sems):
    sem_g = sems[:NB]
    sem_o = sems[NB:]
    wid = lax.axis_index("s") * NC + lax.axis_index("c")
    wbase = wid * RPW
    lane = lax.iota(jnp.int32, L)

    pltpu.sync_copy(idx_hbm.at[pl.ds(wbase, RPW)], idx_v)

    def flatten_chunk(g):
        for u in range(CHUNK // L):
            off = g * CHUNK + u * L
            b = lax.bitwise_and(wbase + off + lane, B - 1)
            idx_v[pl.ds(off, L)] = b * N + idx_v[pl.ds(off, L)]

    gathers = {}
    writes = {}

    def start_gather(g):
        gathers[g] = pltpu.async_copy(
            src_hbm.at[idx_v.at[pl.ds(g * CHUNK, CHUNK)]],
            rows.at[g % NB],
            sem_g[g % NB],
        )

    def start_write(g):
        writes[g] = pltpu.async_copy(
            rows.at[g % NB],
            out_hbm.at[pl.ds(wbase + g * CHUNK, CHUNK)],
            sem_o[g % NB],
        )

    for g in range(NCHUNK):
        flatten_chunk(g)
        if g >= NB:
            writes[g - NB].wait()
        start_gather(g)
        if g >= NB - 1:
            gathers[g - (NB - 1)].wait()
            start_write(g - (NB - 1))
    for g in range(NCHUNK - (NB - 1), NCHUNK):
        gathers[g].wait()
        start_write(g)
    for g in range(NCHUNK - NB, NCHUNK):
        writes[g].wait()


def kernel(source, idxs):
    src = source.reshape(B * N, D)
    idx_kmajor = idxs.astype(jnp.int32)[..., 0].T.reshape(ROWS)
    out = _gather(src, idx_kmajor)
    return out.reshape(K, B, D).transpose(1, 0, 2)
